# baseline (speedup 1.0000x reference)
import functools

import jax
import jax.numpy as jnp
import numpy as np
from jax import lax
from jax.experimental import pallas as pl
from jax.experimental.pallas import tpu as pltpu
from jax.experimental.pallas import tpu_sc as plsc

N = 10000
E = 160000
IN_DIM = 256
HID = 128

NC = 2
NS = 16
NW = NC * NS
EPW = E // NW
CHUNK = 50
CHUNKS = EPW // CHUNK
SLAB = 5
NSLAB = CHUNKS // SLAB
PAIRS = NSLAB // 2
N_PAD = 10240
ROWS_PER_TILE = N_PAD // NS
ZCHUNK = 80

BLK = 2000
GRID = N // BLK
BN_C = float(1.0 / np.sqrt(1.0 + 1e-5))


def _proj(h, wl_ref, wr_ref, g_ref, p_ref, r_ref):
    do = wr_ref.shape[-1]
    w = jnp.concatenate([wl_ref[...], wr_ref[...]], axis=1)
    t = jnp.dot(h, w, preferred_element_type=jnp.float32)
    s = g_ref[...] * BN_C
    pw = p_ref.shape[-1]
    if pw > do:
        pad = jnp.zeros((t.shape[0], pw - do), jnp.float32)
        p_ref[...] = jnp.concatenate([t[:, :do] * s, pad], axis=1)
    else:
        p_ref[...] = t[:, :do] * s
    r_ref[...] = t[:, do:] * s


def _tc_in_body(x_ref, wl_ref, wr_ref, g_ref, p_ref, r_ref):
    _proj(x_ref[...], wl_ref, wr_ref, g_ref, p_ref, r_ref)


def _tc_in(x, wl, wr, g):
    di, do = wl.shape
    return pl.pallas_call(
        _tc_in_body,
        grid=(GRID,),
        in_specs=[
            pl.BlockSpec((BLK, di), lambda i: (i, 0)),
            pl.BlockSpec((di, do), lambda i: (0, 0)),
            pl.BlockSpec((di, do), lambda i: (0, 0)),
            pl.BlockSpec((1, do), lambda i: (0, 0)),
        ],
        out_specs=[
            pl.BlockSpec((BLK, do), lambda i: (i, 0)),
            pl.BlockSpec((BLK, do), lambda i: (i, 0)),
        ],
        out_shape=[
            jax.ShapeDtypeStruct((N, do), jnp.float32),
            jax.ShapeDtypeStruct((N, do), jnp.float32),
        ],
    )(x, wl, wr, g)


def _tc_cnt_body(c_ref, out_ref):
    s = jnp.sum(c_ref[...], axis=0)
    out_ref[...] = (1.0 / jnp.maximum(s, 1.0)).reshape(-1, 1)


def _tc_cnt(cnt_raw):
    return pl.pallas_call(
        _tc_cnt_body,
        grid=(1,),
        in_specs=[pl.BlockSpec((NW, N_PAD), lambda i: (0, 0))],
        out_specs=pl.BlockSpec((N_PAD, 1), lambda i: (0, 0)),
        out_shape=jax.ShapeDtypeStruct((N_PAD, 1), jnp.float32),
    )(cnt_raw)


def _combine(s_ref, c_ref, r_ref, g_ref, bl_ref, b_ref):
    d = r_ref.shape[-1]
    bias = bl_ref[...] * (g_ref[...] * BN_C) + b_ref[...]
    h = ((s_ref[0][:, :d] + s_ref[1][:, :d]) * c_ref[...]
         + r_ref[...] + bias)
    return jnp.maximum(h, 0.0)


def _tc_mid_body(s_ref, c_ref, r_ref, gp_ref, blp_ref, bp_ref,
                 wl_ref, wr_ref, g_ref, p_ref, rout_ref):
    h = _combine(s_ref, c_ref, r_ref, gp_ref, blp_ref, bp_ref)
    _proj(h, wl_ref, wr_ref, g_ref, p_ref, rout_ref)


def _tc_mid(s2, c2, r, gp, blp, bp, wl, wr, g, pad_l):
    di, do = wl.shape
    ds = s2.shape[2]
    return pl.pallas_call(
        _tc_mid_body,
        grid=(GRID,),
        in_specs=[
            pl.BlockSpec((2, BLK, ds), lambda i: (0, i, 0)),
            pl.BlockSpec((BLK, 1), lambda i: (i, 0)),
            pl.BlockSpec((BLK, di), lambda i: (i, 0)),
            pl.BlockSpec((1, di), lambda i: (0, 0)),
            pl.BlockSpec((1, di), lambda i: (0, 0)),
            pl.BlockSpec((1, di), lambda i: (0, 0)),
            pl.BlockSpec((di, do), lambda i: (0, 0)),
            pl.BlockSpec((di, do), lambda i: (0, 0)),
            pl.BlockSpec((1, do), lambda i: (0, 0)),
        ],
        out_specs=[
            pl.BlockSpec((BLK, pad_l), lambda i: (i, 0)),
            pl.BlockSpec((BLK, do), lambda i: (i, 0)),
        ],
        out_shape=[
            jax.ShapeDtypeStruct((N, pad_l), jnp.float32),
            jax.ShapeDtypeStruct((N, do), jnp.float32),
        ],
    )(s2, c2, r, gp, blp, bp, wl, wr, g)


def _tc_head_body(s_ref, c_ref, r_ref, g_ref, bl_ref, b_ref,
                  wh1_ref, bh1_ref, wh2_ref, bh2_ref, out_ref):
    h = _combine(s_ref, c_ref, r_ref, g_ref, bl_ref, b_ref)
    t = jnp.dot(h, wh1_ref[...], preferred_element_type=jnp.float32)
    t = jnp.maximum(t + bh1_ref[...], 0.0)
    out_ref[...] = jnp.dot(t, wh2_ref[...],
                           preferred_element_type=jnp.float32) + bh2_ref[...]


def _tc_head(s2, c2, r, g, bl, b, wh1, bh1, wh2, bh2):
    di = wh1.shape[0]
    ds = s2.shape[2]
    return pl.pallas_call(
        _tc_head_body,
        grid=(GRID,),
        in_specs=[
            pl.BlockSpec((2, BLK, ds), lambda i: (0, i, 0)),
            pl.BlockSpec((BLK, 1), lambda i: (i, 0)),
            pl.BlockSpec((BLK, di), lambda i: (i, 0)),
            pl.BlockSpec((1, di), lambda i: (0, 0)),
            pl.BlockSpec((1, di), lambda i: (0, 0)),
            pl.BlockSpec((1, di), lambda i: (0, 0)),
            pl.BlockSpec((di, 32), lambda i: (0, 0)),
            pl.BlockSpec((1, 32), lambda i: (0, 0)),
            pl.BlockSpec((32, 3), lambda i: (0, 0)),
            pl.BlockSpec((1, 3), lambda i: (0, 0)),
        ],
        out_specs=pl.BlockSpec((BLK, 3), lambda i: (i, 0)),
        out_shape=jax.ShapeDtypeStruct((N, 3), jnp.float32),
    )(s2, c2, r, g, bl, b, wh1, bh1, wh2, bh2)


def _make_edge_agg(d):
    mesh = plsc.VectorSubcoreMesh(core_axis_name="c", subcore_axis_name="s")

    out_type = [jax.ShapeDtypeStruct((NC, N_PAD, d), jnp.float32)]
    scratch = [
        pltpu.VMEM((2 * SLAB, CHUNK), jnp.int32),
        pltpu.VMEM((2 * SLAB, CHUNK), jnp.int32),
        pltpu.VMEM((CHUNK, d), jnp.float32),
        pltpu.VMEM((CHUNK, d), jnp.float32),
        pltpu.VMEM_SHARED((N_PAD, d), jnp.float32),
        pltpu.SemaphoreType.DMA,
        pltpu.SemaphoreType.DMA,
        pltpu.SemaphoreType.DMA,
        pltpu.SemaphoreType.DMA,
        pltpu.SemaphoreType.DMA,
    ]

    def body(p_hbm, idx_hbm, zrow_hbm, out_hbm,
             ib0, ib1, rows0, rows1, acc, isem0, isem1, gsem0, gsem1, zsem):
        cid = lax.axis_index("c")
        sid = lax.axis_index("s")
        wid = sid * NC + cid
        base = sid * ROWS_PER_TILE
        ib = (ib0, ib1)
        rows = (rows0, rows1)
        isem = (isem0, isem1)
        gsem = (gsem0, gsem1)

        def idx_start(t, b):
            pltpu.async_copy(idx_hbm.at[wid, t], ib[b], isem[b])

        def idx_wait(b):
            pltpu.make_async_copy(idx_hbm.at[wid, 0], ib[b], isem[b]).wait()

        def gather_start(sb, row, b):
            pltpu.async_copy(p_hbm.at[ib[sb].at[row]], rows[b], gsem[b])

        def gather_wait(sb, row, b):
            pltpu.make_async_copy(p_hbm.at[ib[sb].at[row]], rows[b],
                                  gsem[b]).wait()

        def scatter(sb, row, b):
            pltpu.sync_copy(rows[b], acc.at[ib[sb].at[row]], add=True)

        zdesc = pltpu.async_copy(
            zrow_hbm, acc.at[pl.ds(base, ROWS_PER_TILE)], zsem)

        idx_start(0, 0)
        idx_wait(0)
        gather_start(0, 0, 0)
        idx_start(1, 1)
        zdesc.wait()
        plsc.subcore_barrier()

        def pair(u, last):
            for k in range(2 * SLAB):
                b = k % 2
                sb = k // SLAB
                srow = 2 * (k % SLAB)
                gather_wait(sb, srow, b)
                if k < 2 * SLAB - 1:
                    nk = k + 1
                    if nk == SLAB:
                        idx_wait(1)
                    gather_start(nk // SLAB, 2 * (nk % SLAB), 1 - b)
                elif not last:
                    idx_wait(0)
                    gather_start(0, 0, 1 - b)
                scatter(sb, srow + 1, b)
                if not last:
                    if k == SLAB - 1:
                        idx_start(2 * u + 2, 0)
                    elif k == 2 * SLAB - 1:
                        idx_start(2 * u + 3, 1)

        def step(u, carry):
            pair(u, False)
            return carry

        lax.fori_loop(0, PAIRS - 1, step, 0)
        pair(PAIRS - 1, True)

        plsc.subcore_barrier()
        pltpu.sync_copy(acc.at[pl.ds(base, ROWS_PER_TILE)],
                        out_hbm.at[cid, pl.ds(base, ROWS_PER_TILE)])

    return pl.kernel(body, out_type=out_type, mesh=mesh,
                     scratch_types=scratch)


def _make_degree():
    mesh = plsc.VectorSubcoreMesh(core_axis_name="c", subcore_axis_name="s")
    out_type = [jax.ShapeDtypeStruct((NW, N_PAD), jnp.float32)]
    scratch = [
        pltpu.VMEM((CHUNKS, CHUNK), jnp.int32),
        pltpu.VMEM((N_PAD,), jnp.float32),
    ]

    def body(dst_hbm, cnt_hbm, dst_v, hist):
        cid = lax.axis_index("c")
        sid = lax.axis_index("s")
        wid = sid * NC + cid

        pltpu.sync_copy(dst_hbm.at[wid], dst_v)
        zeros16 = jnp.zeros((16,), jnp.float32)

        def zero(i, carry):
            hist[pl.ds(i * 16, 16)] = zeros16
            return carry

        lax.fori_loop(0, N_PAD // 16, zero, 0)

        ones16 = jnp.ones((16,), jnp.float32)
        tail = CHUNK % 16
        tail_mask = lax.iota(jnp.int32, 16) >= (16 - tail)

        def row(j, carry):
            for k in range(CHUNK // 16):
                idx = dst_v[j, pl.ds(k * 16, 16)]
                plsc.addupdate_scatter(hist, [idx], ones16)
            if tail:
                idx = dst_v[j, pl.ds(CHUNK - 16, 16)]
                plsc.addupdate_scatter(hist, [idx], ones16, mask=tail_mask)
            return carry

        lax.fori_loop(0, CHUNKS, row, 0)
        pltpu.sync_copy(hist, cnt_hbm.at[wid])

    return pl.kernel(
        body, out_type=out_type, mesh=mesh, scratch_types=scratch,
        compiler_params=pltpu.CompilerParams(needs_layout_passes=False))


@functools.lru_cache(maxsize=None)
def _edge_agg(d):
    return _make_edge_agg(d)


@functools.lru_cache(maxsize=None)
def _degree():
    return _make_degree()


def kernel(x, edge_index, params):
    def row(name):
        return params[name].reshape(1, -1)

    src = edge_index[0].reshape(NW, NSLAB, SLAB, CHUNK)
    dst = edge_index[1].reshape(NW, NSLAB, SLAB, CHUNK)
    idx_all = jnp.stack([src, dst], axis=3).reshape(NW, NSLAB,
                                                    2 * SLAB, CHUNK)
    z128 = jnp.zeros((ROWS_PER_TILE, HID), jnp.float32)

    (cnt_raw,) = _degree()(edge_index[1].reshape(NW, CHUNKS, CHUNK))
    cnt = _tc_cnt(cnt_raw)
    p1, r1 = _tc_in(x, params["Wl1"], params["Wr1"], row("g1"))
    (s1,) = _edge_agg(HID)(p1, idx_all, z128)
    p2, r2 = _tc_mid(s1, cnt, r1, row("g1"), row("bl1"), row("b1"),
                     params["Wl2"], params["Wr2"], row("g2"), HID)
    (s2,) = _edge_agg(HID)(p2, idx_all, z128)
    p3, r3 = _tc_mid(s2, cnt, r2, row("g2"), row("bl2"), row("b2"),
                     params["Wl3"], params["Wr3"], row("g3"), HID)
    (s3,) = _edge_agg(HID)(p3, idx_all, z128)
    out = _tc_head(s3, cnt, r3, row("g3"), row("bl3"), row("b3"),
                   params["Wh1"], row("bh1"), params["Wh2"], row("bh2"))
    return out


# speedup vs baseline: 1.3765x; 1.3765x over previous
import functools

import jax
import jax.numpy as jnp
import numpy as np
from jax import lax
from jax.experimental import pallas as pl
from jax.experimental.pallas import tpu as pltpu
from jax.experimental.pallas import tpu_sc as plsc

N = 10000
E = 160000
IN_DIM = 256
HID = 128

NC = 2
NS = 16
NW = NC * NS
EPW = E // NW
CHUNK = 125
CHUNKS = EPW // CHUNK
SLAB = 5
NSLAB = CHUNKS // SLAB
PAIRS = NSLAB // 2
N_PAD = 10240
ROWS_PER_TILE = N_PAD // NS
ZCHUNK = 80

BLK = 2000
GRID = N // BLK
BN_C = float(1.0 / np.sqrt(1.0 + 1e-5))


def _proj(h, wl_ref, wr_ref, g_ref, p_ref, r_ref):
    do = wr_ref.shape[-1]
    w = jnp.concatenate([wl_ref[...], wr_ref[...]], axis=1)
    t = jnp.dot(h, w, preferred_element_type=jnp.float32)
    s = g_ref[...] * BN_C
    pw = p_ref.shape[-1]
    if pw > do:
        pad = jnp.zeros((t.shape[0], pw - do), jnp.float32)
        p_ref[...] = jnp.concatenate([t[:, :do] * s, pad], axis=1)
    else:
        p_ref[...] = t[:, :do] * s
    r_ref[...] = t[:, do:] * s


def _tc_in_body(x_ref, wl_ref, wr_ref, g_ref, p_ref, r_ref):
    _proj(x_ref[...], wl_ref, wr_ref, g_ref, p_ref, r_ref)


def _tc_in(x, wl, wr, g):
    di, do = wl.shape
    return pl.pallas_call(
        _tc_in_body,
        grid=(GRID,),
        in_specs=[
            pl.BlockSpec((BLK, di), lambda i: (i, 0)),
            pl.BlockSpec((di, do), lambda i: (0, 0)),
            pl.BlockSpec((di, do), lambda i: (0, 0)),
            pl.BlockSpec((1, do), lambda i: (0, 0)),
        ],
        out_specs=[
            pl.BlockSpec((BLK, do), lambda i: (i, 0)),
            pl.BlockSpec((BLK, do), lambda i: (i, 0)),
        ],
        out_shape=[
            jax.ShapeDtypeStruct((N, do), jnp.float32),
            jax.ShapeDtypeStruct((N, do), jnp.float32),
        ],
    )(x, wl, wr, g)


def _tc_cnt_body(c_ref, out_ref):
    s = jnp.sum(c_ref[...], axis=0)
    out_ref[...] = (1.0 / jnp.maximum(s, 1.0)).reshape(-1, 1)


def _tc_cnt(cnt_raw):
    return pl.pallas_call(
        _tc_cnt_body,
        grid=(1,),
        in_specs=[pl.BlockSpec((NW, N_PAD), lambda i: (0, 0))],
        out_specs=pl.BlockSpec((N_PAD, 1), lambda i: (0, 0)),
        out_shape=jax.ShapeDtypeStruct((N_PAD, 1), jnp.float32),
    )(cnt_raw)


def _combine(s_ref, c_ref, r_ref, g_ref, bl_ref, b_ref):
    d = r_ref.shape[-1]
    bias = bl_ref[...] * (g_ref[...] * BN_C) + b_ref[...]
    h = ((s_ref[0][:, :d] + s_ref[1][:, :d]) * c_ref[...]
         + r_ref[...] + bias)
    return jnp.maximum(h, 0.0)


def _tc_mid_body(s_ref, c_ref, r_ref, gp_ref, blp_ref, bp_ref,
                 wl_ref, wr_ref, g_ref, p_ref, rout_ref):
    h = _combine(s_ref, c_ref, r_ref, gp_ref, blp_ref, bp_ref)
    _proj(h, wl_ref, wr_ref, g_ref, p_ref, rout_ref)


def _tc_mid(s2, c2, r, gp, blp, bp, wl, wr, g, pad_l):
    di, do = wl.shape
    ds = s2.shape[2]
    return pl.pallas_call(
        _tc_mid_body,
        grid=(GRID,),
        in_specs=[
            pl.BlockSpec((2, BLK, ds), lambda i: (0, i, 0)),
            pl.BlockSpec((BLK, 1), lambda i: (i, 0)),
            pl.BlockSpec((BLK, di), lambda i: (i, 0)),
            pl.BlockSpec((1, di), lambda i: (0, 0)),
            pl.BlockSpec((1, di), lambda i: (0, 0)),
            pl.BlockSpec((1, di), lambda i: (0, 0)),
            pl.BlockSpec((di, do), lambda i: (0, 0)),
            pl.BlockSpec((di, do), lambda i: (0, 0)),
            pl.BlockSpec((1, do), lambda i: (0, 0)),
        ],
        out_specs=[
            pl.BlockSpec((BLK, pad_l), lambda i: (i, 0)),
            pl.BlockSpec((BLK, do), lambda i: (i, 0)),
        ],
        out_shape=[
            jax.ShapeDtypeStruct((N, pad_l), jnp.float32),
            jax.ShapeDtypeStruct((N, do), jnp.float32),
        ],
    )(s2, c2, r, gp, blp, bp, wl, wr, g)


def _tc_head_body(s_ref, c_ref, r_ref, g_ref, bl_ref, b_ref,
                  wh1_ref, bh1_ref, wh2_ref, bh2_ref, out_ref):
    h = _combine(s_ref, c_ref, r_ref, g_ref, bl_ref, b_ref)
    t = jnp.dot(h, wh1_ref[...], preferred_element_type=jnp.float32)
    t = jnp.maximum(t + bh1_ref[...], 0.0)
    out_ref[...] = jnp.dot(t, wh2_ref[...],
                           preferred_element_type=jnp.float32) + bh2_ref[...]


def _tc_head(s2, c2, r, g, bl, b, wh1, bh1, wh2, bh2):
    di = wh1.shape[0]
    ds = s2.shape[2]
    return pl.pallas_call(
        _tc_head_body,
        grid=(GRID,),
        in_specs=[
            pl.BlockSpec((2, BLK, ds), lambda i: (0, i, 0)),
            pl.BlockSpec((BLK, 1), lambda i: (i, 0)),
            pl.BlockSpec((BLK, di), lambda i: (i, 0)),
            pl.BlockSpec((1, di), lambda i: (0, 0)),
            pl.BlockSpec((1, di), lambda i: (0, 0)),
            pl.BlockSpec((1, di), lambda i: (0, 0)),
            pl.BlockSpec((di, 32), lambda i: (0, 0)),
            pl.BlockSpec((1, 32), lambda i: (0, 0)),
            pl.BlockSpec((32, 3), lambda i: (0, 0)),
            pl.BlockSpec((1, 3), lambda i: (0, 0)),
        ],
        out_specs=pl.BlockSpec((BLK, 3), lambda i: (i, 0)),
        out_shape=jax.ShapeDtypeStruct((N, 3), jnp.float32),
    )(s2, c2, r, g, bl, b, wh1, bh1, wh2, bh2)


def _make_edge_agg(d):
    mesh = plsc.VectorSubcoreMesh(core_axis_name="c", subcore_axis_name="s")

    out_type = [jax.ShapeDtypeStruct((NC, N_PAD, d), jnp.float32)]
    scratch = [
        pltpu.VMEM((2 * SLAB, CHUNK), jnp.int32),
        pltpu.VMEM((2 * SLAB, CHUNK), jnp.int32),
        pltpu.VMEM((CHUNK, d), jnp.float32),
        pltpu.VMEM((CHUNK, d), jnp.float32),
        pltpu.VMEM_SHARED((N_PAD, d), jnp.float32),
        pltpu.SemaphoreType.DMA,
        pltpu.SemaphoreType.DMA,
        pltpu.SemaphoreType.DMA,
        pltpu.SemaphoreType.DMA,
        pltpu.SemaphoreType.DMA,
    ]

    def body(p_hbm, idx_hbm, zrow_hbm, out_hbm,
             ib0, ib1, rows0, rows1, acc, isem0, isem1, gsem0, gsem1, zsem):
        cid = lax.axis_index("c")
        sid = lax.axis_index("s")
        wid = sid * NC + cid
        base = sid * ROWS_PER_TILE
        ib = (ib0, ib1)
        rows = (rows0, rows1)
        isem = (isem0, isem1)
        gsem = (gsem0, gsem1)

        def idx_start(t, b):
            pltpu.async_copy(idx_hbm.at[wid, t], ib[b], isem[b])

        def idx_wait(b):
            pltpu.make_async_copy(idx_hbm.at[wid, 0], ib[b], isem[b]).wait()

        def gather_start(sb, row, b):
            pltpu.async_copy(p_hbm.at[ib[sb].at[row]], rows[b], gsem[b])

        def gather_wait(sb, row, b):
            pltpu.make_async_copy(p_hbm.at[ib[sb].at[row]], rows[b],
                                  gsem[b]).wait()

        def scatter(sb, row, b):
            pltpu.sync_copy(rows[b], acc.at[ib[sb].at[row]], add=True)

        zdesc = pltpu.async_copy(
            zrow_hbm, acc.at[pl.ds(base, ROWS_PER_TILE)], zsem)

        idx_start(0, 0)
        idx_wait(0)
        gather_start(0, 0, 0)
        idx_start(1, 1)
        zdesc.wait()
        plsc.subcore_barrier()

        def pair(u, last):
            for k in range(2 * SLAB):
                b = k % 2
                sb = k // SLAB
                srow = 2 * (k % SLAB)
                gather_wait(sb, srow, b)
                if k < 2 * SLAB - 1:
                    nk = k + 1
                    if nk == SLAB:
                        idx_wait(1)
                    gather_start(nk // SLAB, 2 * (nk % SLAB), 1 - b)
                elif not last:
                    idx_wait(0)
                    gather_start(0, 0, 1 - b)
                scatter(sb, srow + 1, b)
                if not last:
                    if k == SLAB - 1:
                        idx_start(2 * u + 2, 0)
                    elif k == 2 * SLAB - 1:
                        idx_start(2 * u + 3, 1)

        def step(u, carry):
            pair(u, False)
            return carry

        lax.fori_loop(0, PAIRS - 1, step, 0)
        pair(PAIRS - 1, True)

        plsc.subcore_barrier()
        pltpu.sync_copy(acc.at[pl.ds(base, ROWS_PER_TILE)],
                        out_hbm.at[cid, pl.ds(base, ROWS_PER_TILE)])

    return pl.kernel(body, out_type=out_type, mesh=mesh,
                     scratch_types=scratch)


def _make_degree():
    mesh = plsc.VectorSubcoreMesh(core_axis_name="c", subcore_axis_name="s")
    out_type = [jax.ShapeDtypeStruct((NW, N_PAD), jnp.float32)]
    scratch = [
        pltpu.VMEM((CHUNKS, CHUNK), jnp.int32),
        pltpu.VMEM((N_PAD,), jnp.float32),
    ]

    def body(dst_hbm, cnt_hbm, dst_v, hist):
        cid = lax.axis_index("c")
        sid = lax.axis_index("s")
        wid = sid * NC + cid

        pltpu.sync_copy(dst_hbm.at[wid], dst_v)
        zeros16 = jnp.zeros((16,), jnp.float32)

        def zero(i, carry):
            hist[pl.ds(i * 16, 16)] = zeros16
            return carry

        lax.fori_loop(0, N_PAD // 16, zero, 0)

        ones16 = jnp.ones((16,), jnp.float32)
        tail = CHUNK % 16
        tail_mask = lax.iota(jnp.int32, 16) >= (16 - tail)

        def row(j, carry):
            for k in range(CHUNK // 16):
                idx = dst_v[j, pl.ds(k * 16, 16)]
                plsc.addupdate_scatter(hist, [idx], ones16)
            if tail:
                idx = dst_v[j, pl.ds(CHUNK - 16, 16)]
                plsc.addupdate_scatter(hist, [idx], ones16, mask=tail_mask)
            return carry

        lax.fori_loop(0, CHUNKS, row, 0)
        pltpu.sync_copy(hist, cnt_hbm.at[wid])

    return pl.kernel(
        body, out_type=out_type, mesh=mesh, scratch_types=scratch,
        compiler_params=pltpu.CompilerParams(needs_layout_passes=False))


@functools.lru_cache(maxsize=None)
def _edge_agg(d):
    return _make_edge_agg(d)


@functools.lru_cache(maxsize=None)
def _degree():
    return _make_degree()


def kernel(x, edge_index, params):
    def row(name):
        return params[name].reshape(1, -1)

    src = edge_index[0].reshape(NW, NSLAB, SLAB, CHUNK)
    dst = edge_index[1].reshape(NW, NSLAB, SLAB, CHUNK)
    idx_all = jnp.stack([src, dst], axis=3).reshape(NW, NSLAB,
                                                    2 * SLAB, CHUNK)
    z128 = jnp.zeros((ROWS_PER_TILE, HID), jnp.float32)

    (cnt_raw,) = _degree()(edge_index[1].reshape(NW, CHUNKS, CHUNK))
    cnt = _tc_cnt(cnt_raw)
    p1, r1 = _tc_in(x, params["Wl1"], params["Wr1"], row("g1"))
    (s1,) = _edge_agg(HID)(p1, idx_all, z128)
    p2, r2 = _tc_mid(s1, cnt, r1, row("g1"), row("bl1"), row("b1"),
                     params["Wl2"], params["Wr2"], row("g2"), HID)
    (s2,) = _edge_agg(HID)(p2, idx_all, z128)
    p3, r3 = _tc_mid(s2, cnt, r2, row("g2"), row("bl2"), row("b2"),
                     params["Wl3"], params["Wr3"], row("g3"), HID)
    (s3,) = _edge_agg(HID)(p3, idx_all, z128)
    out = _tc_head(s3, cnt, r3, row("g3"), row("bl3"), row("b3"),
                   params["Wh1"], row("bh1"), params["Wh2"], row("bh2"))
    return out


# speedup vs baseline: 1.4030x; 1.0193x over previous
import functools

import jax
import jax.numpy as jnp
import numpy as np
from jax import lax
from jax.experimental import pallas as pl
from jax.experimental.pallas import tpu as pltpu
from jax.experimental.pallas import tpu_sc as plsc

N = 10000
E = 160000
IN_DIM = 256
HID = 128

NC = 2
NS = 16
NW = NC * NS
EPW = E // NW
CHUNK = 125
CHUNKS = EPW // CHUNK
SLAB = 5
NSLAB = CHUNKS // SLAB
PAIRS = NSLAB // 2
N_PAD = 10240
ROWS_PER_TILE = N_PAD // NS
ZCHUNK = 80

BLK = 5000
GRID = N // BLK
BN_C = float(1.0 / np.sqrt(1.0 + 1e-5))


def _proj(h, wl_ref, wr_ref, g_ref, p_ref, r_ref):
    do = wr_ref.shape[-1]
    w = jnp.concatenate([wl_ref[...], wr_ref[...]], axis=1)
    t = jnp.dot(h, w, preferred_element_type=jnp.float32)
    s = g_ref[...] * BN_C
    pw = p_ref.shape[-1]
    if pw > do:
        pad = jnp.zeros((t.shape[0], pw - do), jnp.float32)
        p_ref[...] = jnp.concatenate([t[:, :do] * s, pad], axis=1)
    else:
        p_ref[...] = t[:, :do] * s
    r_ref[...] = t[:, do:] * s


def _tc_in_body(x_ref, wl_ref, wr_ref, g_ref, p_ref, r_ref):
    _proj(x_ref[...], wl_ref, wr_ref, g_ref, p_ref, r_ref)


def _tc_in(x, wl, wr, g):
    di, do = wl.shape
    return pl.pallas_call(
        _tc_in_body,
        grid=(GRID,),
        in_specs=[
            pl.BlockSpec((BLK, di), lambda i: (i, 0)),
            pl.BlockSpec((di, do), lambda i: (0, 0)),
            pl.BlockSpec((di, do), lambda i: (0, 0)),
            pl.BlockSpec((1, do), lambda i: (0, 0)),
        ],
        out_specs=[
            pl.BlockSpec((BLK, do), lambda i: (i, 0)),
            pl.BlockSpec((BLK, do), lambda i: (i, 0)),
        ],
        out_shape=[
            jax.ShapeDtypeStruct((N, do), jnp.float32),
            jax.ShapeDtypeStruct((N, do), jnp.float32),
        ],
    )(x, wl, wr, g)


def _tc_cnt_body(c_ref, out_ref):
    s = jnp.sum(c_ref[...], axis=0)
    out_ref[...] = (1.0 / jnp.maximum(s, 1.0)).reshape(-1, 1)


def _tc_cnt(cnt_raw):
    return pl.pallas_call(
        _tc_cnt_body,
        grid=(1,),
        in_specs=[pl.BlockSpec((NW, N_PAD), lambda i: (0, 0))],
        out_specs=pl.BlockSpec((N_PAD, 1), lambda i: (0, 0)),
        out_shape=jax.ShapeDtypeStruct((N_PAD, 1), jnp.float32),
    )(cnt_raw)


def _combine(s_ref, c_ref, r_ref, g_ref, bl_ref, b_ref):
    d = r_ref.shape[-1]
    bias = bl_ref[...] * (g_ref[...] * BN_C) + b_ref[...]
    h = ((s_ref[0][:, :d] + s_ref[1][:, :d]) * c_ref[...]
         + r_ref[...] + bias)
    return jnp.maximum(h, 0.0)


def _tc_mid_body(s_ref, c_ref, r_ref, gp_ref, blp_ref, bp_ref,
                 wl_ref, wr_ref, g_ref, p_ref, rout_ref):
    h = _combine(s_ref, c_ref, r_ref, gp_ref, blp_ref, bp_ref)
    _proj(h, wl_ref, wr_ref, g_ref, p_ref, rout_ref)


def _tc_mid(s2, c2, r, gp, blp, bp, wl, wr, g, pad_l):
    di, do = wl.shape
    ds = s2.shape[2]
    return pl.pallas_call(
        _tc_mid_body,
        grid=(GRID,),
        in_specs=[
            pl.BlockSpec((2, BLK, ds), lambda i: (0, i, 0)),
            pl.BlockSpec((BLK, 1), lambda i: (i, 0)),
            pl.BlockSpec((BLK, di), lambda i: (i, 0)),
            pl.BlockSpec((1, di), lambda i: (0, 0)),
            pl.BlockSpec((1, di), lambda i: (0, 0)),
            pl.BlockSpec((1, di), lambda i: (0, 0)),
            pl.BlockSpec((di, do), lambda i: (0, 0)),
            pl.BlockSpec((di, do), lambda i: (0, 0)),
            pl.BlockSpec((1, do), lambda i: (0, 0)),
        ],
        out_specs=[
            pl.BlockSpec((BLK, pad_l), lambda i: (i, 0)),
            pl.BlockSpec((BLK, do), lambda i: (i, 0)),
        ],
        out_shape=[
            jax.ShapeDtypeStruct((N, pad_l), jnp.float32),
            jax.ShapeDtypeStruct((N, do), jnp.float32),
        ],
    )(s2, c2, r, gp, blp, bp, wl, wr, g)


def _tc_head_body(s_ref, c_ref, r_ref, g_ref, bl_ref, b_ref,
                  wh1_ref, bh1_ref, wh2_ref, bh2_ref, out_ref):
    h = _combine(s_ref, c_ref, r_ref, g_ref, bl_ref, b_ref)
    t = jnp.dot(h, wh1_ref[...], preferred_element_type=jnp.float32)
    t = jnp.maximum(t + bh1_ref[...], 0.0)
    out_ref[...] = jnp.dot(t, wh2_ref[...],
                           preferred_element_type=jnp.float32) + bh2_ref[...]


def _tc_head(s2, c2, r, g, bl, b, wh1, bh1, wh2, bh2):
    di = wh1.shape[0]
    ds = s2.shape[2]
    return pl.pallas_call(
        _tc_head_body,
        grid=(GRID,),
        in_specs=[
            pl.BlockSpec((2, BLK, ds), lambda i: (0, i, 0)),
            pl.BlockSpec((BLK, 1), lambda i: (i, 0)),
            pl.BlockSpec((BLK, di), lambda i: (i, 0)),
            pl.BlockSpec((1, di), lambda i: (0, 0)),
            pl.BlockSpec((1, di), lambda i: (0, 0)),
            pl.BlockSpec((1, di), lambda i: (0, 0)),
            pl.BlockSpec((di, 32), lambda i: (0, 0)),
            pl.BlockSpec((1, 32), lambda i: (0, 0)),
            pl.BlockSpec((32, 3), lambda i: (0, 0)),
            pl.BlockSpec((1, 3), lambda i: (0, 0)),
        ],
        out_specs=pl.BlockSpec((BLK, 3), lambda i: (i, 0)),
        out_shape=jax.ShapeDtypeStruct((N, 3), jnp.float32),
    )(s2, c2, r, g, bl, b, wh1, bh1, wh2, bh2)


def _make_edge_agg(d):
    mesh = plsc.VectorSubcoreMesh(core_axis_name="c", subcore_axis_name="s")

    out_type = [jax.ShapeDtypeStruct((NC, N_PAD, d), jnp.float32)]
    scratch = [
        pltpu.VMEM((2 * SLAB, CHUNK), jnp.int32),
        pltpu.VMEM((2 * SLAB, CHUNK), jnp.int32),
        pltpu.VMEM((CHUNK, d), jnp.float32),
        pltpu.VMEM((CHUNK, d), jnp.float32),
        pltpu.VMEM_SHARED((N_PAD, d), jnp.float32),
        pltpu.SemaphoreType.DMA,
        pltpu.SemaphoreType.DMA,
        pltpu.SemaphoreType.DMA,
        pltpu.SemaphoreType.DMA,
        pltpu.SemaphoreType.DMA,
    ]

    def body(p_hbm, idx_hbm, zrow_hbm, out_hbm,
             ib0, ib1, rows0, rows1, acc, isem0, isem1, gsem0, gsem1, zsem):
        cid = lax.axis_index("c")
        sid = lax.axis_index("s")
        wid = sid * NC + cid
        base = sid * ROWS_PER_TILE
        ib = (ib0, ib1)
        rows = (rows0, rows1)
        isem = (isem0, isem1)
        gsem = (gsem0, gsem1)

        def idx_start(t, b):
            pltpu.async_copy(idx_hbm.at[wid, t], ib[b], isem[b])

        def idx_wait(b):
            pltpu.make_async_copy(idx_hbm.at[wid, 0], ib[b], isem[b]).wait()

        def gather_start(sb, row, b):
            pltpu.async_copy(p_hbm.at[ib[sb].at[row]], rows[b], gsem[b])

        def gather_wait(sb, row, b):
            pltpu.make_async_copy(p_hbm.at[ib[sb].at[row]], rows[b],
                                  gsem[b]).wait()

        def scatter(sb, row, b):
            pltpu.sync_copy(rows[b], acc.at[ib[sb].at[row]], add=True)

        zdesc = pltpu.async_copy(
            zrow_hbm, acc.at[pl.ds(base, ROWS_PER_TILE)], zsem)

        idx_start(0, 0)
        idx_wait(0)
        gather_start(0, 0, 0)
        idx_start(1, 1)
        zdesc.wait()
        plsc.subcore_barrier()

        def pair(u, last):
            for k in range(2 * SLAB):
                b = k % 2
                sb = k // SLAB
                srow = 2 * (k % SLAB)
                gather_wait(sb, srow, b)
                if k < 2 * SLAB - 1:
                    nk = k + 1
                    if nk == SLAB:
                        idx_wait(1)
                    gather_start(nk // SLAB, 2 * (nk % SLAB), 1 - b)
                elif not last:
                    idx_wait(0)
                    gather_start(0, 0, 1 - b)
                scatter(sb, srow + 1, b)
                if not last:
                    if k == SLAB - 1:
                        idx_start(2 * u + 2, 0)
                    elif k == 2 * SLAB - 1:
                        idx_start(2 * u + 3, 1)

        def step(u, carry):
            pair(u, False)
            return carry

        lax.fori_loop(0, PAIRS - 1, step, 0)
        pair(PAIRS - 1, True)

        plsc.subcore_barrier()
        pltpu.sync_copy(acc.at[pl.ds(base, ROWS_PER_TILE)],
                        out_hbm.at[cid, pl.ds(base, ROWS_PER_TILE)])

    return pl.kernel(body, out_type=out_type, mesh=mesh,
                     scratch_types=scratch)


def _make_degree():
    mesh = plsc.VectorSubcoreMesh(core_axis_name="c", subcore_axis_name="s")
    out_type = [jax.ShapeDtypeStruct((NW, N_PAD), jnp.float32)]
    scratch = [
        pltpu.VMEM((CHUNKS, CHUNK), jnp.int32),
        pltpu.VMEM((N_PAD,), jnp.float32),
    ]

    def body(dst_hbm, cnt_hbm, dst_v, hist):
        cid = lax.axis_index("c")
        sid = lax.axis_index("s")
        wid = sid * NC + cid

        pltpu.sync_copy(dst_hbm.at[wid], dst_v)
        zeros16 = jnp.zeros((16,), jnp.float32)

        def zero(i, carry):
            hist[pl.ds(i * 16, 16)] = zeros16
            return carry

        lax.fori_loop(0, N_PAD // 16, zero, 0)

        ones16 = jnp.ones((16,), jnp.float32)
        tail = CHUNK % 16
        tail_mask = lax.iota(jnp.int32, 16) >= (16 - tail)

        def row(j, carry):
            for k in range(CHUNK // 16):
                idx = dst_v[j, pl.ds(k * 16, 16)]
                plsc.addupdate_scatter(hist, [idx], ones16)
            if tail:
                idx = dst_v[j, pl.ds(CHUNK - 16, 16)]
                plsc.addupdate_scatter(hist, [idx], ones16, mask=tail_mask)
            return carry

        lax.fori_loop(0, CHUNKS, row, 0)
        pltpu.sync_copy(hist, cnt_hbm.at[wid])

    return pl.kernel(
        body, out_type=out_type, mesh=mesh, scratch_types=scratch,
        compiler_params=pltpu.CompilerParams(needs_layout_passes=False))


@functools.lru_cache(maxsize=None)
def _edge_agg(d):
    return _make_edge_agg(d)


@functools.lru_cache(maxsize=None)
def _degree():
    return _make_degree()


def kernel(x, edge_index, params):
    def row(name):
        return params[name].reshape(1, -1)

    src = edge_index[0].reshape(NW, NSLAB, SLAB, CHUNK)
    dst = edge_index[1].reshape(NW, NSLAB, SLAB, CHUNK)
    idx_all = jnp.stack([src, dst], axis=3).reshape(NW, NSLAB,
                                                    2 * SLAB, CHUNK)
    z128 = jnp.zeros((ROWS_PER_TILE, HID), jnp.float32)

    (cnt_raw,) = _degree()(edge_index[1].reshape(NW, CHUNKS, CHUNK))
    cnt = _tc_cnt(cnt_raw)
    p1, r1 = _tc_in(x, params["Wl1"], params["Wr1"], row("g1"))
    (s1,) = _edge_agg(HID)(p1, idx_all, z128)
    p2, r2 = _tc_mid(s1, cnt, r1, row("g1"), row("bl1"), row("b1"),
                     params["Wl2"], params["Wr2"], row("g2"), HID)
    (s2,) = _edge_agg(HID)(p2, idx_all, z128)
    p3, r3 = _tc_mid(s2, cnt, r2, row("g2"), row("bl2"), row("b2"),
                     params["Wl3"], params["Wr3"], row("g3"), HID)
    (s3,) = _edge_agg(HID)(p3, idx_all, z128)
    out = _tc_head(s3, cnt, r3, row("g3"), row("bl3"), row("b3"),
                   params["Wh1"], row("bh1"), params["Wh2"], row("bh2"))
    return out
